# baseline (speedup 1.0000x reference)
import functools

import jax
import jax.numpy as jnp
from jax.experimental import pallas as pl
from jax.experimental.pallas import tpu as pltpu

_ALPHA = 0.99
_C = 1.0 - _ALPHA
_INIT_HI = -60.0
_INIT_LO = -90.0
_VAR0 = 40.0 ** 2


def _t_chunk(t: int, f: int, bl: int) -> int:
    budget = min(100, 4 * 1024 * 1024 // (f * bl * 4))
    best = 1
    for s in range(1, t + 1):
        if t % s == 0 and s <= budget:
            best = s
    return best


def _body(st, f, bl, x_ref, o_ref, mu_sc, v2_sc):
    j = pl.program_id(1)

    @pl.when(j == 0)
    def _init():
        frow = jax.lax.broadcasted_iota(jnp.int32, (f, bl), 0)
        step = (_INIT_LO - _INIT_HI) / (f - 1)
        mu_sc[...] = _INIT_HI + frow.astype(jnp.float32) * step
        v2_sc[...] = jnp.full((f, bl), _VAR0 / (_ALPHA * _ALPHA),
                              dtype=jnp.float32)

    mu = mu_sc[...]
    v2 = v2_sc[...]
    for idx in range(st):
        xv = x_ref[idx]
        e = xv - mu
        mu = mu + _C * e
        v2 = _ALPHA * v2 + _C * (e * e)
        o_ref[idx] = e * jax.lax.rsqrt(v2)
    mu_sc[...] = mu
    v2_sc[...] = v2


def kernel(x):
    b, t_total, f = x.shape
    xt = jnp.transpose(x, (1, 2, 0))
    bl = 128 if b % 128 == 0 else b
    nb = b // bl
    st = _t_chunk(t_total, f, bl)
    nt = t_total // st

    body = functools.partial(_body, st, f, bl)
    out_t = pl.pallas_call(
        body,
        grid=(nb, nt),
        in_specs=[pl.BlockSpec((st, f, bl), lambda i, j: (j, 0, i))],
        out_specs=pl.BlockSpec((st, f, bl), lambda i, j: (j, 0, i)),
        out_shape=jax.ShapeDtypeStruct((t_total, f, b), jnp.float32),
        scratch_shapes=[
            pltpu.VMEM((f, bl), jnp.float32),
            pltpu.VMEM((f, bl), jnp.float32),
        ],
        compiler_params=pltpu.CompilerParams(
            dimension_semantics=("parallel", "arbitrary"),
        ),
    )(xt)
    return jnp.transpose(out_t, (2, 0, 1))


# speedup vs baseline: 1.0761x; 1.0761x over previous
import functools

import jax
import jax.numpy as jnp
from jax.experimental import pallas as pl
from jax.experimental.pallas import tpu as pltpu

_ALPHA = 0.99
_C = 1.0 - _ALPHA
_INIT_HI = -60.0
_INIT_LO = -90.0
_VAR0 = 40.0 ** 2


def _t_chunk(t: int, f: int, bl: int) -> int:
    budget = min(256, 8 * 1024 * 1024 // (f * bl * 4))
    best = 1
    for s in range(1, t + 1):
        if t % s == 0 and s <= budget:
            best = s
    return best


def _body(st, f, bl, x_ref, o_ref, mu_sc, v2_sc):
    j = pl.program_id(1)

    @pl.when(j == 0)
    def _init():
        frow = jax.lax.broadcasted_iota(jnp.int32, (f, bl), 0)
        step = (_INIT_LO - _INIT_HI) / (f - 1)
        mu_sc[...] = _INIT_HI + frow.astype(jnp.float32) * step
        v2_sc[...] = jnp.full((f, bl), _VAR0 / (_ALPHA * _ALPHA),
                              dtype=jnp.float32)

    mu = mu_sc[...]
    v2 = v2_sc[...]
    for idx in range(st):
        xv = x_ref[idx]
        e = xv - mu
        mu = mu + _C * e
        v2 = _ALPHA * v2 + _C * (e * e)
        o_ref[idx] = e * jax.lax.rsqrt(v2)
    mu_sc[...] = mu
    v2_sc[...] = v2


def kernel(x):
    b, t_total, f = x.shape
    xt = jnp.transpose(x, (1, 2, 0))
    bl = 128 if b % 128 == 0 else b
    nb = b // bl
    st = _t_chunk(t_total, f, bl)
    nt = t_total // st

    body = functools.partial(_body, st, f, bl)
    out_t = pl.pallas_call(
        body,
        grid=(nb, nt),
        in_specs=[pl.BlockSpec((st, f, bl), lambda i, j: (j, 0, i))],
        out_specs=pl.BlockSpec((st, f, bl), lambda i, j: (j, 0, i)),
        out_shape=jax.ShapeDtypeStruct((t_total, f, b), jnp.float32),
        scratch_shapes=[
            pltpu.VMEM((f, bl), jnp.float32),
            pltpu.VMEM((f, bl), jnp.float32),
        ],
        compiler_params=pltpu.CompilerParams(
            dimension_semantics=("parallel", "arbitrary"),
        ),
    )(xt)
    return jnp.transpose(out_t, (2, 0, 1))


# speedup vs baseline: 1.0804x; 1.0040x over previous
import functools

import jax
import jax.numpy as jnp
from jax.experimental import pallas as pl
from jax.experimental.pallas import tpu as pltpu

_ALPHA = 0.99
_C = 1.0 - _ALPHA
_INIT_HI = -60.0
_INIT_LO = -90.0
_VAR0 = 40.0 ** 2


def _t_chunk(t: int, f: int, bl: int) -> int:
    budget = min(500, 13 * 1024 * 1024 // (f * bl * 4))
    best = 1
    for s in range(1, t + 1):
        if t % s == 0 and s <= budget:
            best = s
    return best


def _body(st, f, bl, x_ref, o_ref, mu_sc, v2_sc):
    j = pl.program_id(1)

    @pl.when(j == 0)
    def _init():
        frow = jax.lax.broadcasted_iota(jnp.int32, (f, bl), 0)
        step = (_INIT_LO - _INIT_HI) / (f - 1)
        mu_sc[...] = _INIT_HI + frow.astype(jnp.float32) * step
        v2_sc[...] = jnp.full((f, bl), _VAR0 / (_ALPHA * _ALPHA),
                              dtype=jnp.float32)

    mu = mu_sc[...]
    v2 = v2_sc[...]
    for idx in range(st):
        xv = x_ref[idx]
        e = xv - mu
        mu = mu + _C * e
        v2 = _ALPHA * v2 + _C * (e * e)
        o_ref[idx] = e * jax.lax.rsqrt(v2)
    mu_sc[...] = mu
    v2_sc[...] = v2


def kernel(x):
    b, t_total, f = x.shape
    xt = jnp.transpose(x, (1, 2, 0))
    bl = 128 if b % 128 == 0 else b
    nb = b // bl
    st = _t_chunk(t_total, f, bl)
    nt = t_total // st

    body = functools.partial(_body, st, f, bl)
    out_t = pl.pallas_call(
        body,
        grid=(nb, nt),
        in_specs=[pl.BlockSpec((st, f, bl), lambda i, j: (j, 0, i))],
        out_specs=pl.BlockSpec((st, f, bl), lambda i, j: (j, 0, i)),
        out_shape=jax.ShapeDtypeStruct((t_total, f, b), jnp.float32),
        scratch_shapes=[
            pltpu.VMEM((f, bl), jnp.float32),
            pltpu.VMEM((f, bl), jnp.float32),
        ],
        compiler_params=pltpu.CompilerParams(
            dimension_semantics=("parallel", "arbitrary"),
        ),
    )(xt)
    return jnp.transpose(out_t, (2, 0, 1))
